# baseline (speedup 1.0000x reference)
import functools

import jax
import jax.numpy as jnp
from jax import lax
from jax.experimental import pallas as pl
from jax.experimental.pallas import tpu as pltpu
from jax.experimental.pallas import tpu_sc as plsc

VOCAB = 1000000
EMBED_DIM = 64
BATCH = 16384

_NUM_CORES = 2
_NUM_SUBCORES = 16
_NUM_WORKERS = _NUM_CORES * _NUM_SUBCORES
_NUM_PGROUPS = 16
_B_PER_W = BATCH // _NUM_PGROUPS
_DIMS_PER_W = EMBED_DIM // 2
_K = 16
_WIN = 128


@functools.partial(
    pl.kernel,
    mesh=plsc.VectorSubcoreMesh(core_axis_name="c", subcore_axis_name="s"),
    out_type=jax.ShapeDtypeStruct((EMBED_DIM, BATCH), jnp.float32),
    scratch_types=[
        pltpu.VMEM((_B_PER_W,), jnp.int32),
        pltpu.VMEM((_K, _DIMS_PER_W, _WIN), jnp.float32),
        pltpu.VMEM((_DIMS_PER_W, _B_PER_W), jnp.float32),
        pltpu.SemaphoreType.DMA,
        pltpu.SemaphoreType.DMA,
    ],
    compiler_params=pltpu.CompilerParams(
        use_tc_tiling_on_sc=True, needs_layout_passes=False
    ),
)
def _sc_gather(idx_hbm, tablet_hbm, outt_hbm, idx_v, win_v, outc_v, isem, sem):
    wid = lax.axis_index("s") * _NUM_CORES + lax.axis_index("c")
    pg = lax.shift_right_logical(wid, 1)
    half = lax.bitwise_and(wid, 1)
    base = pl.multiple_of(pg * _B_PER_W, _B_PER_W)
    dbase = pl.multiple_of(half * _DIMS_PER_W, _DIMS_PER_W)
    pltpu.sync_copy(idx_hbm.at[pl.ds(base, _B_PER_W)], idx_v)
    lane = lax.iota(jnp.int32, 16)

    def fire(i, t):
        c0 = pl.multiple_of(
            lax.shift_left(lax.shift_right_logical(i, 7), 7), _WIN
        )
        pltpu.async_copy(
            tablet_hbm.at[pl.ds(dbase, _DIMS_PER_W), pl.ds(c0, _WIN)],
            win_v.at[t],
            sem,
        )

    def extract(i, j, t):
        pltpu.make_async_copy(
            tablet_hbm.at[pl.ds(0, _DIMS_PER_W), pl.ds(0, _WIN)],
            win_v.at[t],
            sem,
        ).wait()
        col = jnp.broadcast_to(lax.bitwise_and(i, _WIN - 1), (16,))
        pos = jnp.broadcast_to(j, (16,))
        for k in range(_DIMS_PER_W // 16):
            row = lane + 16 * k
            vals = plsc.load_gather(win_v.at[t], [row, col])
            plsc.store_scatter(outc_v, [row, pos], vals)

    vec0 = idx_v[pl.ds(0, 16)]
    for t in range(_K):
        fire(vec0[t], t)

    def body(g, prev_vec):
        vec = idx_v[pl.ds(g * 16, 16)]
        for t in range(_K):
            extract(prev_vec[t], (g - 1) * 16 + t, t)
            fire(vec[t], t)
        return vec

    vec_last = lax.fori_loop(1, _B_PER_W // 16, body, vec0)
    for t in range(_K):
        extract(vec_last[t], _B_PER_W - 16 + t, t)

    pltpu.async_copy(
        outc_v,
        outt_hbm.at[pl.ds(dbase, _DIMS_PER_W), pl.ds(base, _B_PER_W)],
        isem,
    ).wait()


def kernel(X, table0):
    idx = X.reshape(BATCH).astype(jnp.int32)
    return _sc_gather(idx, table0.T).T


# speedup vs baseline: 1.0121x; 1.0121x over previous
import functools

import jax
import jax.numpy as jnp
from jax import lax
from jax.experimental import pallas as pl
from jax.experimental.pallas import tpu as pltpu
from jax.experimental.pallas import tpu_sc as plsc

VOCAB = 1000000
EMBED_DIM = 64
BATCH = 16384

_NUM_CORES = 2
_NUM_SUBCORES = 16
_NUM_WORKERS = _NUM_CORES * _NUM_SUBCORES
_B_PER_W = BATCH // _NUM_WORKERS
_K = 8
_WIN = 128


@functools.partial(
    pl.kernel,
    mesh=plsc.VectorSubcoreMesh(core_axis_name="c", subcore_axis_name="s"),
    out_type=jax.ShapeDtypeStruct((EMBED_DIM, BATCH), jnp.float32),
    scratch_types=[
        pltpu.VMEM((_B_PER_W,), jnp.int32),
        pltpu.VMEM((_K, EMBED_DIM, _WIN), jnp.float32),
        pltpu.VMEM((EMBED_DIM, _B_PER_W), jnp.float32),
        pltpu.SemaphoreType.DMA,
        pltpu.SemaphoreType.DMA,
    ],
    compiler_params=pltpu.CompilerParams(
        use_tc_tiling_on_sc=True, needs_layout_passes=False
    ),
)
def _sc_gather(idx_hbm, tablet_hbm, outt_hbm, idx_v, win_v, outc_v, isem, sem):
    wid = lax.axis_index("s") * _NUM_CORES + lax.axis_index("c")
    base = pl.multiple_of(wid * _B_PER_W, _B_PER_W)
    pltpu.sync_copy(idx_hbm.at[pl.ds(base, _B_PER_W)], idx_v)
    lane = lax.iota(jnp.int32, 16)

    def fire(i, t):
        c0 = pl.multiple_of(
            lax.shift_left(lax.shift_right_logical(i, 7), 7), _WIN
        )
        pltpu.async_copy(tablet_hbm.at[:, pl.ds(c0, _WIN)], win_v.at[t], sem)

    def extract(i, j, t):
        pltpu.make_async_copy(
            tablet_hbm.at[:, pl.ds(0, _WIN)], win_v.at[t], sem
        ).wait()
        col = jnp.broadcast_to(lax.bitwise_and(i, _WIN - 1), (16,))
        pos = jnp.broadcast_to(j, (16,))
        for k in range(EMBED_DIM // 16):
            row = lane + 16 * k
            vals = plsc.load_gather(win_v.at[t], [row, col])
            plsc.store_scatter(outc_v, [row, pos], vals)

    vec0 = idx_v[pl.ds(0, 16)]
    for t in range(_K):
        fire(vec0[t], t)

    def body(g, prev_vec):
        vec = idx_v[pl.ds(g * 16, 16)]
        for t in range(_K):
            extract(prev_vec[_K + t], (g - 1) * 16 + _K + t, t)
            fire(vec[t], t)
        for t in range(_K):
            extract(vec[t], g * 16 + t, t)
            fire(vec[_K + t], t)
        return vec

    def first(g, prev_vec):
        for t in range(_K):
            extract(prev_vec[t], t, t)
            fire(prev_vec[_K + t], t)
        return prev_vec

    vec_last = lax.fori_loop(1, _B_PER_W // 16, body, first(0, vec0))
    for t in range(_K):
        extract(vec_last[_K + t], _B_PER_W - _K + t, t)

    pltpu.async_copy(outc_v, outt_hbm.at[:, pl.ds(base, _B_PER_W)], isem).wait()


def kernel(X, table0):
    idx = X.reshape(BATCH).astype(jnp.int32)
    return _sc_gather(idx, table0.T).T
